# baseline (speedup 1.0000x reference)
import functools

import jax
import jax.numpy as jnp
from jax import lax
from jax.experimental import pallas as pl
from jax.experimental.pallas import tpu as pltpu
from jax.experimental.pallas import tpu_sc as plsc

NC = 2
NS = 16
L = 16
NW = NC * NS


def _matmul_kernel(p_ref, d_ref, out_ref):
    a_hw = (d_ref[...] == 1).astype(jnp.float32)
    out_ref[...] = jnp.dot(p_ref[...], a_hw, preferred_element_type=jnp.float32)


def _compute_pa(p_flat, d_hw, block_rows):
    rows = p_flat.shape[0]
    q = p_flat.shape[1]
    grid = rows // block_rows
    return pl.pallas_call(
        _matmul_kernel,
        grid=(grid,),
        in_specs=[
            pl.BlockSpec((block_rows, q), lambda i: (i, 0)),
            pl.BlockSpec((q, q), lambda i: (0, 0)),
        ],
        out_specs=pl.BlockSpec((block_rows, q), lambda i: (i, 0)),
        out_shape=jax.ShapeDtypeStruct((rows, q), jnp.float32),
    )(p_flat, d_hw)


def _finalize_kernel(s_ref, w_ref, o_ref, *, wps, b):
    total = jnp.float32(0.0)
    for bb in range(b):
        sb = jnp.sum(s_ref[bb * wps:(bb + 1) * wps, :])
        wb = jnp.maximum(jnp.sum(w_ref[bb * wps:(bb + 1) * wps, :]), 1e-8)
        total = total + sb / wb
    o_ref[0, 0] = -total / b


def _make_sc_kernel(q, e, wps, cs):
    qc = q // L
    qw = q // 2
    totc = e // cs
    mesh = plsc.VectorSubcoreMesh(
        core_axis_name="c", subcore_axis_name="s", num_cores=NC, num_subcores=NS)

    def body(pa_hbm, p_hbm, i_hbm, j_hbm, w_hbm, s_out, w_out,
             iv0, iv1, jv0, jv1, wv0, wv1, ri0, ri1, rj0, rj1,
             stage, s_ii, s_jj, s_ww, s_ri, s_rj):
        cid = lax.axis_index("c")
        sid = lax.axis_index("s")
        wid = sid * NC + cid
        b = wid // wps
        r = wid % wps
        half = totc // 2
        h0 = (r * half) // wps
        cnt = 2 * (((r + 1) * half) // wps - h0)
        base = b * e + 2 * h0 * cs

        ivs = (iv0, iv1)
        jvs = (jv0, jv1)
        wvs = (wv0, wv1)
        ris = (ri0, ri1)
        rjs = (rj0, rj1)

        def fire_idx(k, slot):
            off = base + k * cs
            pltpu.async_copy(i_hbm.at[pl.ds(off, cs)], ivs[slot], s_ii)
            pltpu.async_copy(j_hbm.at[pl.ds(off, cs)], jvs[slot], s_jj)
            pltpu.async_copy(w_hbm.at[pl.ds(off, cs)], wvs[slot], s_ww)

        def wait_idx(slot):
            pltpu.make_async_copy(i_hbm.at[pl.ds(0, cs)], ivs[slot], s_ii).wait()
            pltpu.make_async_copy(j_hbm.at[pl.ds(0, cs)], jvs[slot], s_jj).wait()
            pltpu.make_async_copy(w_hbm.at[pl.ds(0, cs)], wvs[slot], s_ww).wait()

        def fire_rows(slot):
            pltpu.async_copy(pa_hbm.at[ivs[slot]], ris[slot], s_ri)
            pltpu.async_copy(p_hbm.at[jvs[slot]], rjs[slot], s_rj)

        def wait_rows(slot):
            pltpu.make_async_copy(pa_hbm.at[ivs[slot]], ris[slot], s_ri).wait()
            pltpu.make_async_copy(p_hbm.at[jvs[slot]], rjs[slot], s_rj).wait()

        def accum_chunk(slot, carry):
            ri_s = ris[slot]
            rj_s = rjs[slot]
            wv_s = wvs[slot]

            def group_body(g, carry):
                accs, wacc = carry
                w16 = wv_s[pl.ds(g * L, L)]
                wacc = wacc + w16
                for k in range(L):
                    ee = g * L + k
                    wspl = w16[k]
                    parts = []
                    for c in range(qc):
                        pi = ri_s[ee, pl.ds(c * L, L)]
                        pj = rj_s[ee, pl.ds(c * L, L)]
                        parts.append(pi * pj)
                    s0 = (parts[0] + parts[1]) + (parts[2] + parts[3])
                    s1 = (parts[4] + parts[5]) + (parts[6] + parts[7])
                    accs = accs + (s0 + s1) * wspl
                return accs, wacc

            return plsc.parallel_loop(0, cs // L, unroll=2, carry=carry)(group_body)

        zero = jnp.zeros((L,), jnp.float32)
        carry0 = (zero, zero)

        fire_idx(0, 0)
        wait_idx(0)
        fire_rows(0)
        fire_idx(jnp.where(cnt > 1, 1, 0), 1)

        def half_step(k, slot, carry):
            pslot = 1 - slot
            wait_idx(slot)
            wait_rows(pslot)
            fire_rows(slot)
            fire_idx(jnp.where(k + 1 < cnt, k + 1, 0), pslot)
            return accum_chunk(pslot, carry)

        def pair_body(kk, carry):
            k = 1 + 2 * kk
            carry = half_step(k, 1, carry)
            carry = half_step(k + 1, 0, carry)
            return carry

        carry = lax.fori_loop(0, cnt // 2, pair_body, carry0)
        wait_idx(1)
        wait_rows(0)

        stot, wacc = carry
        stage[pl.ds(0, L)] = stot
        stage[pl.ds(L, L)] = wacc
        pltpu.sync_copy(stage.at[pl.ds(0, L)], s_out.at[pl.ds(wid * L, L)])
        pltpu.sync_copy(stage.at[pl.ds(L, L)], w_out.at[pl.ds(wid * L, L)])

    return pl.kernel(
        body,
        out_type=(
            jax.ShapeDtypeStruct((NW * L,), jnp.float32),
            jax.ShapeDtypeStruct((NW * L,), jnp.float32),
        ),
        mesh=mesh,
        scratch_types=[
            pltpu.VMEM((cs,), jnp.int32),
            pltpu.VMEM((cs,), jnp.int32),
            pltpu.VMEM((cs,), jnp.int32),
            pltpu.VMEM((cs,), jnp.int32),
            pltpu.VMEM((cs,), jnp.float32),
            pltpu.VMEM((cs,), jnp.float32),
            pltpu.VMEM((cs, q), jnp.float32),
            pltpu.VMEM((cs, q), jnp.float32),
            pltpu.VMEM((cs, q), jnp.float32),
            pltpu.VMEM((cs, q), jnp.float32),
            pltpu.VMEM((2 * L,), jnp.float32),
            pltpu.SemaphoreType.DMA,
            pltpu.SemaphoreType.DMA,
            pltpu.SemaphoreType.DMA,
            pltpu.SemaphoreType.DMA,
            pltpu.SemaphoreType.DMA,
        ],
    )


def kernel(P, d_hw, circuit_edge_pairs, circuit_edge_weights):
    b, n, q = P.shape
    e = circuit_edge_pairs.shape[1]

    p_flat = P.reshape(b * n, q)
    offs = (jnp.arange(b, dtype=jnp.int32) * n)[:, None]
    i_flat = (circuit_edge_pairs[:, :, 0] + offs).reshape(b * e)
    j_flat = (circuit_edge_pairs[:, :, 1] + offs).reshape(b * e)
    w_flat = circuit_edge_weights.reshape(b * e)

    pa_flat = _compute_pa(p_flat, d_hw, block_rows=1000)

    pa_pk = pa_flat
    p_pk = p_flat

    wps = NW // b
    cs = 128
    sc = _make_sc_kernel(q, e, wps, cs)
    s_part, w_part = sc(pa_pk, p_pk, i_flat, j_flat, w_flat)
    s_part = s_part.reshape(NW, L)
    w_part = w_part.reshape(NW, L)

    fin = pl.pallas_call(
        functools.partial(_finalize_kernel, wps=wps, b=b),
        in_specs=[
            pl.BlockSpec(memory_space=pltpu.VMEM),
            pl.BlockSpec(memory_space=pltpu.VMEM),
        ],
        out_specs=pl.BlockSpec(memory_space=pltpu.SMEM),
        out_shape=jax.ShapeDtypeStruct((1, 1), jnp.float32),
    )(s_part, w_part)
    return fin[0, 0]


# speedup vs baseline: 1.9593x; 1.9593x over previous
import functools

import jax
import jax.numpy as jnp
from jax import lax
from jax.experimental import pallas as pl
from jax.experimental.pallas import tpu as pltpu
from jax.experimental.pallas import tpu_sc as plsc

NC = 2
NS = 16
L = 16
NW = NC * NS


def _matmul_kernel(p_ref, d_ref, out_ref):
    a_hw = (d_ref[...] == 1).astype(jnp.float32)
    out_ref[...] = jnp.dot(p_ref[...], a_hw, preferred_element_type=jnp.float32)


def _compute_pa(p_flat, d_hw, block_rows):
    rows = p_flat.shape[0]
    q = p_flat.shape[1]
    grid = rows // block_rows
    return pl.pallas_call(
        _matmul_kernel,
        grid=(grid,),
        in_specs=[
            pl.BlockSpec((block_rows, q), lambda i: (i, 0)),
            pl.BlockSpec((q, q), lambda i: (0, 0)),
        ],
        out_specs=pl.BlockSpec((block_rows, q), lambda i: (i, 0)),
        out_shape=jax.ShapeDtypeStruct((rows, q), jnp.float32),
    )(p_flat, d_hw)


def _finalize_kernel(s_ref, w_ref, o_ref, *, wps, b):
    total = jnp.float32(0.0)
    for bb in range(b):
        sb = jnp.sum(s_ref[bb * wps:(bb + 1) * wps, :])
        wb = jnp.maximum(jnp.sum(w_ref[bb * wps:(bb + 1) * wps, :]), 1e-8)
        total = total + sb / wb
    o_ref[0, 0] = -total / b


def _make_sc_kernel(q, e, wps, cs):
    qc = q // L
    qw = q // 2
    totc = e // cs
    mesh = plsc.VectorSubcoreMesh(
        core_axis_name="c", subcore_axis_name="s", num_cores=NC, num_subcores=NS)

    def body(pa_hbm, p_hbm, i_hbm, j_hbm, w_hbm, s_out, w_out,
             iv0, iv1, iv2, iv3, jv0, jv1, jv2, jv3, wv0, wv1, wv2, wv3,
             ri0, ri1, rj0, rj1, stage,
             s_i0, s_i1, s_i2, s_i3, s_ri0, s_ri1, s_rj0, s_rj1):
        cid = lax.axis_index("c")
        sid = lax.axis_index("s")
        wid = sid * NC + cid
        b = wid // wps
        r = wid % wps
        quarter = totc // 4
        q0 = (r * quarter) // wps
        cnt = 4 * (((r + 1) * quarter) // wps - q0)
        base = b * e + 4 * q0 * cs

        ivs = (iv0, iv1, iv2, iv3)
        jvs = (jv0, jv1, jv2, jv3)
        wvs = (wv0, wv1, wv2, wv3)
        ris = (ri0, ri1)
        rjs = (rj0, rj1)
        sis = (s_i0, s_i1, s_i2, s_i3)
        sris = (s_ri0, s_ri1)
        srjs = (s_rj0, s_rj1)

        def fire_idx(k, slot):
            off = base + k * cs
            pltpu.async_copy(i_hbm.at[pl.ds(off, cs)], ivs[slot], sis[slot])
            pltpu.async_copy(j_hbm.at[pl.ds(off, cs)], jvs[slot], sis[slot])
            pltpu.async_copy(w_hbm.at[pl.ds(off, cs)], wvs[slot], sis[slot])

        def wait_idx(slot):
            pltpu.make_async_copy(i_hbm.at[pl.ds(0, cs)], ivs[slot], sis[slot]).wait()
            pltpu.make_async_copy(j_hbm.at[pl.ds(0, cs)], jvs[slot], sis[slot]).wait()
            pltpu.make_async_copy(w_hbm.at[pl.ds(0, cs)], wvs[slot], sis[slot]).wait()

        def fire_rows(slot, islot):
            pltpu.async_copy(pa_hbm.at[ivs[islot]], ris[slot], sris[slot])
            pltpu.async_copy(p_hbm.at[jvs[islot]], rjs[slot], srjs[slot])

        def wait_rows(slot, islot):
            pltpu.make_async_copy(pa_hbm.at[ivs[islot]], ris[slot], sris[slot]).wait()
            pltpu.make_async_copy(p_hbm.at[jvs[islot]], rjs[slot], srjs[slot]).wait()

        def accum_chunk(slot, islot, carry):
            ri_s = ris[slot]
            rj_s = rjs[slot]
            wv_s = wvs[islot]

            def group_body(g, carry):
                accs, wacc = carry
                w16 = wv_s[pl.ds(g * L, L)]
                wacc = wacc + w16
                for k in range(L):
                    ee = g * L + k
                    wspl = w16[k]
                    parts = []
                    for cw in range(qc // 2):
                        bi = plsc.bitcast(ri_s[ee, pl.ds(cw * L, L)], jnp.bfloat16)
                        bj = plsc.bitcast(rj_s[ee, pl.ds(cw * L, L)], jnp.bfloat16)
                        lo, hi = plsc.unpack(
                            bi * bj, format=plsc.PackFormat.INTERLEAVED)
                        parts.append(lo + hi)
                    tot = (parts[0] + parts[1]) + (parts[2] + parts[3])
                    accs = accs + tot * wspl
                return accs, wacc

            return plsc.parallel_loop(0, cs // L, unroll=2, carry=carry)(group_body)

        zero = jnp.zeros((L,), jnp.float32)
        carry0 = (zero, zero)

        fire_idx(0, 0)
        wait_idx(0)
        fire_rows(0, 0)
        fire_idx(jnp.where(cnt > 1, 1, 0), 1)

        def step(k, i4, carry):
            r2 = i4 % 2
            p4 = (i4 - 1) % 4
            pr2 = 1 - r2
            wait_idx(i4)
            fire_rows(r2, i4)
            fire_idx(jnp.where(k + 1 < cnt, k + 1, 0), (i4 + 1) % 4)
            wait_rows(pr2, p4)
            return accum_chunk(pr2, p4, carry)

        def quad_body(kk, carry):
            k = 1 + 4 * kk
            for d in range(4):
                carry = step(k + d, (1 + d) % 4, carry)
            return carry

        carry = lax.fori_loop(0, cnt // 4, quad_body, carry0)
        wait_idx(1)
        wait_rows(0, 0)

        stot, wacc = carry
        stage[pl.ds(0, L)] = stot
        stage[pl.ds(L, L)] = wacc
        pltpu.sync_copy(stage.at[pl.ds(0, L)], s_out.at[pl.ds(wid * L, L)])
        pltpu.sync_copy(stage.at[pl.ds(L, L)], w_out.at[pl.ds(wid * L, L)])

    return pl.kernel(
        body,
        out_type=(
            jax.ShapeDtypeStruct((NW * L,), jnp.float32),
            jax.ShapeDtypeStruct((NW * L,), jnp.float32),
        ),
        mesh=mesh,
        compiler_params=pltpu.CompilerParams(
            needs_layout_passes=False, use_tc_tiling_on_sc=False),
        scratch_types=(
            [pltpu.VMEM((cs,), jnp.int32)] * 8
            + [pltpu.VMEM((cs,), jnp.float32)] * 4
            + [pltpu.VMEM((cs, qw), jnp.int32)] * 4
            + [pltpu.VMEM((2 * L,), jnp.float32)]
            + [pltpu.SemaphoreType.DMA] * 8
        ),
    )


def kernel(P, d_hw, circuit_edge_pairs, circuit_edge_weights):
    b, n, q = P.shape
    e = circuit_edge_pairs.shape[1]

    p_flat = P.reshape(b * n, q)
    offs = (jnp.arange(b, dtype=jnp.int32) * n)[:, None]
    i_flat = (circuit_edge_pairs[:, :, 0] + offs).reshape(b * e)
    j_flat = (circuit_edge_pairs[:, :, 1] + offs).reshape(b * e)
    w_flat = circuit_edge_weights.reshape(b * e)

    pa_flat = _compute_pa(p_flat, d_hw, block_rows=1000)

    pa_pk = jax.lax.bitcast_convert_type(
        pa_flat.astype(jnp.bfloat16).reshape(b * n, q // 2, 2), jnp.int32)
    p_pk = jax.lax.bitcast_convert_type(
        p_flat.astype(jnp.bfloat16).reshape(b * n, q // 2, 2), jnp.int32)

    wps = NW // b
    cs = 128
    sc = _make_sc_kernel(q, e, wps, cs)
    s_part, w_part = sc(pa_pk, p_pk, i_flat, j_flat, w_flat)
    s_part = s_part.reshape(NW, L)
    w_part = w_part.reshape(NW, L)

    fin = pl.pallas_call(
        functools.partial(_finalize_kernel, wps=wps, b=b),
        in_specs=[
            pl.BlockSpec(memory_space=pltpu.VMEM),
            pl.BlockSpec(memory_space=pltpu.VMEM),
        ],
        out_specs=pl.BlockSpec(memory_space=pltpu.SMEM),
        out_shape=jax.ShapeDtypeStruct((1, 1), jnp.float32),
    )(s_part, w_part)
    return fin[0, 0]
